# baseline (speedup 1.0000x reference)
import functools

import jax
import jax.numpy as jnp
from jax import lax
from jax.experimental import pallas as pl
from jax.experimental.pallas import tpu as pltpu
from jax.experimental.pallas import tpu_sc as plsc

EMB_D = 1024


@functools.cache
def _make_lookup(B: int, D: int):
    info = plsc.get_sparse_core_info()
    NC, NS = info.num_cores, info.num_subcores
    NW = NC * NS
    assert B % NW == 0
    b_per_w = B // NW
    C = 8
    NBUF = 8
    AHEAD = 7
    n_chunks = b_per_w // C
    n_outer = n_chunks // NBUF
    assert b_per_w % C == 0 and n_chunks % NBUF == 0 and AHEAD < NBUF

    mesh = plsc.VectorSubcoreMesh(core_axis_name="c", subcore_axis_name="s")

    @functools.partial(
        pl.kernel,
        mesh=mesh,
        out_type=jax.ShapeDtypeStruct((B, D), jnp.float32),
        scratch_types=[
            pltpu.VMEM((b_per_w,), jnp.int32),
            *[pltpu.VMEM((C, D), jnp.float32) for _ in range(NBUF)],
            *[pltpu.SemaphoreType.DMA for _ in range(2 * NBUF)],
        ],
    )
    def lookup(idx_hbm, table_hbm, out_hbm, idx_v, *bufs_sems):
        bufs = bufs_sems[:NBUF]
        gsems = bufs_sems[NBUF : 2 * NBUF]
        wsems = bufs_sems[2 * NBUF :]
        wid = lax.axis_index("s") * NC + lax.axis_index("c")
        base = wid * b_per_w
        pltpu.sync_copy(idx_hbm.at[pl.ds(base, b_per_w)], idx_v)

        def fire_gather(i, b):
            return pltpu.async_copy(
                table_hbm.at[idx_v.at[pl.ds(i * C, C)]], bufs[b], gsems[b]
            )

        def fire_writeback(i, b):
            return pltpu.async_copy(
                bufs[b], out_hbm.at[pl.ds(base + i * C, C)], wsems[b]
            )

        def wait_gather(b):
            pltpu.make_async_copy(
                table_hbm.at[idx_v.at[pl.ds(0, C)]], bufs[b], gsems[b]
            ).wait()

        def wait_writeback(b):
            pltpu.make_async_copy(
                bufs[b], out_hbm.at[pl.ds(base, C)], wsems[b]
            ).wait()

        S = NBUF - AHEAD
        for j in range(AHEAD):
            fire_gather(j, j % NBUF)

        def outer(g, carry):
            t0 = g * NBUF
            for b in range(NBUF):
                t = t0 + b
                wait_gather(b)
                fire_writeback(t, b)
                rb = (b + AHEAD) % NBUF
                rx = t + AHEAD

                @pl.when(rx < n_chunks)
                def _():
                    @pl.when(t >= S)
                    def _():
                        wait_writeback(rb)

                    fire_gather(rx, rb)

            return carry

        lax.fori_loop(0, n_outer, outer, 0)
        for b in range(NBUF):
            wait_writeback(b)

    return lookup


def kernel(input_ids, embedding_table):
    input_shape = input_ids.shape
    flat_ids = input_ids.reshape(-1).astype(jnp.int32)
    out = _make_lookup(flat_ids.shape[0], EMB_D)(flat_ids, embedding_table)
    return (out.reshape(input_shape + (EMB_D,)), embedding_table)


# speedup vs baseline: 1.0002x; 1.0002x over previous
import functools

import jax
import jax.numpy as jnp
from jax import lax
from jax.experimental import pallas as pl
from jax.experimental.pallas import tpu as pltpu
from jax.experimental.pallas import tpu_sc as plsc

EMB_D = 1024


@functools.cache
def _make_lookup(B: int, D: int):
    info = plsc.get_sparse_core_info()
    NC, NS = info.num_cores, info.num_subcores
    NW = NC * NS
    assert B % NW == 0
    b_per_w = B // NW
    C = 8
    NBUF = 8
    AHEAD = 4
    n_chunks = b_per_w // C
    n_outer = n_chunks // NBUF
    assert b_per_w % C == 0 and n_chunks % NBUF == 0 and AHEAD < NBUF

    mesh = plsc.VectorSubcoreMesh(core_axis_name="c", subcore_axis_name="s")

    @functools.partial(
        pl.kernel,
        mesh=mesh,
        out_type=jax.ShapeDtypeStruct((B, D), jnp.float32),
        scratch_types=[
            pltpu.VMEM((b_per_w,), jnp.int32),
            *[pltpu.VMEM((C, D), jnp.float32) for _ in range(NBUF)],
            *[pltpu.SemaphoreType.DMA for _ in range(2 * NBUF)],
        ],
    )
    def lookup(idx_hbm, table_hbm, out_hbm, idx_v, *bufs_sems):
        bufs = bufs_sems[:NBUF]
        gsems = bufs_sems[NBUF : 2 * NBUF]
        wsems = bufs_sems[2 * NBUF :]
        wid = lax.axis_index("s") * NC + lax.axis_index("c")
        base = wid * b_per_w
        pltpu.sync_copy(idx_hbm.at[pl.ds(base, b_per_w)], idx_v)

        def fire_gather(i, b):
            return pltpu.async_copy(
                table_hbm.at[idx_v.at[pl.ds(i * C, C)]], bufs[b], gsems[b]
            )

        def fire_writeback(i, b):
            return pltpu.async_copy(
                bufs[b], out_hbm.at[pl.ds(base + i * C, C)], wsems[b]
            )

        def wait_gather(b):
            pltpu.make_async_copy(
                table_hbm.at[idx_v.at[pl.ds(0, C)]], bufs[b], gsems[b]
            ).wait()

        def wait_writeback(b):
            pltpu.make_async_copy(
                bufs[b], out_hbm.at[pl.ds(base, C)], wsems[b]
            ).wait()

        S = NBUF - AHEAD
        for j in range(AHEAD):
            fire_gather(j, j % NBUF)

        def outer(g, carry):
            t0 = g * NBUF
            for b in range(NBUF):
                t = t0 + b
                wait_gather(b)
                fire_writeback(t, b)
                rb = (b + AHEAD) % NBUF
                rx = t + AHEAD

                @pl.when(rx < n_chunks)
                def _():
                    @pl.when(t >= S)
                    def _():
                        wait_writeback(rb)

                    fire_gather(rx, rb)

            return carry

        lax.fori_loop(0, n_outer, outer, 0)
        for b in range(NBUF):
            wait_writeback(b)

    return lookup


def kernel(input_ids, embedding_table):
    input_shape = input_ids.shape
    flat_ids = input_ids.reshape(-1).astype(jnp.int32)
    out = _make_lookup(flat_ids.shape[0], EMB_D)(flat_ids, embedding_table)
    return (out.reshape(input_shape + (EMB_D,)), embedding_table)


# speedup vs baseline: 1.0014x; 1.0012x over previous
import functools

import jax
import jax.numpy as jnp
from jax import lax
from jax.experimental import pallas as pl
from jax.experimental.pallas import tpu as pltpu
from jax.experimental.pallas import tpu_sc as plsc

EMB_D = 1024


@functools.cache
def _make_lookup(B: int, D: int):
    info = plsc.get_sparse_core_info()
    NC, NS = info.num_cores, info.num_subcores
    NW = NC * NS
    assert B % NW == 0
    b_per_w = B // NW
    C = 8
    NBUF = 8
    AHEAD = 6
    n_chunks = b_per_w // C
    n_outer = n_chunks // NBUF
    assert b_per_w % C == 0 and n_chunks % NBUF == 0 and AHEAD < NBUF

    mesh = plsc.VectorSubcoreMesh(core_axis_name="c", subcore_axis_name="s")

    @functools.partial(
        pl.kernel,
        mesh=mesh,
        out_type=jax.ShapeDtypeStruct((B, D), jnp.float32),
        scratch_types=[
            pltpu.VMEM((b_per_w,), jnp.int32),
            *[pltpu.VMEM((C, D), jnp.float32) for _ in range(NBUF)],
            *[pltpu.SemaphoreType.DMA for _ in range(2 * NBUF)],
        ],
    )
    def lookup(idx_hbm, table_hbm, out_hbm, idx_v, *bufs_sems):
        bufs = bufs_sems[:NBUF]
        gsems = bufs_sems[NBUF : 2 * NBUF]
        wsems = bufs_sems[2 * NBUF :]
        wid = lax.axis_index("s") * NC + lax.axis_index("c")
        base = wid * b_per_w
        pltpu.sync_copy(idx_hbm.at[pl.ds(base, b_per_w)], idx_v)

        def fire_gather(i, b):
            return pltpu.async_copy(
                table_hbm.at[idx_v.at[pl.ds(i * C, C)]], bufs[b], gsems[b]
            )

        def fire_writeback(i, b):
            return pltpu.async_copy(
                bufs[b], out_hbm.at[pl.ds(base + i * C, C)], wsems[b]
            )

        def wait_gather(b):
            pltpu.make_async_copy(
                table_hbm.at[idx_v.at[pl.ds(0, C)]], bufs[b], gsems[b]
            ).wait()

        def wait_writeback(b):
            pltpu.make_async_copy(
                bufs[b], out_hbm.at[pl.ds(base, C)], wsems[b]
            ).wait()

        S = NBUF - AHEAD
        for j in range(AHEAD):
            fire_gather(j, j % NBUF)

        def outer(g, carry):
            t0 = g * NBUF
            for b in range(NBUF):
                t = t0 + b
                wait_gather(b)
                fire_writeback(t, b)
                rb = (b + AHEAD) % NBUF
                rx = t + AHEAD

                @pl.when(rx < n_chunks)
                def _():
                    @pl.when(t >= S)
                    def _():
                        wait_writeback(rb)

                    fire_gather(rx, rb)

            return carry

        lax.fori_loop(0, n_outer, outer, 0)
        for b in range(NBUF):
            wait_writeback(b)

    return lookup


def kernel(input_ids, embedding_table):
    input_shape = input_ids.shape
    flat_ids = input_ids.reshape(-1).astype(jnp.int32)
    out = _make_lookup(flat_ids.shape[0], EMB_D)(flat_ids, embedding_table)
    return (out.reshape(input_shape + (EMB_D,)), embedding_table)
